# Initial kernel scaffold; baseline (speedup 1.0000x reference)
import jax
import jax.numpy as jnp
from jax.experimental import pallas as pl


def kernel(x, edge_index, W1, b1, Wmu, bmu, Wls, bls):
    raise NotImplementedError("write your pallas kernel here")


# speedup vs baseline: 14.2459x; 14.2459x over previous
import functools

import jax
import jax.numpy as jnp
from jax import lax
from jax.experimental import pallas as pl
from jax.experimental.pallas import tpu as pltpu
from jax.experimental.pallas import tpu_sc as plsc

_N = 10000
_E = 320000
_NSUB = 16
_NCORE = 2
_NW = _NCORE * _NSUB
_CHUNK = 128
_CHUNKS = -(-_E // (_NW * _CHUNK))
_EPAD = _NW * _CHUNKS * _CHUNK
_NPAD = 10112
_RPT = _NPAD // _NSUB
_DW = 128

_mesh = plsc.VectorSubcoreMesh(core_axis_name="c", subcore_axis_name="s")


def _sc_degree_body(dst_hbm, zeros_hbm, ones_hbm, out_hbm,
                    dst_v, ones_v, deg_sh):
    cid = lax.axis_index("c")
    sid = lax.axis_index("s")
    wid = cid * _NSUB + sid
    row0 = sid * _RPT
    pltpu.sync_copy(zeros_hbm.at[pl.ds(row0, _RPT)],
                    deg_sh.at[pl.ds(row0, _RPT)])
    pltpu.sync_copy(dst_hbm.at[wid], dst_v)
    pltpu.sync_copy(ones_hbm, ones_v)
    plsc.subcore_barrier()

    def step(j, carry):
        pltpu.sync_copy(ones_v, deg_sh.at[dst_v.at[j]], add=True)
        return carry

    lax.fori_loop(0, _CHUNKS, step, 0)
    plsc.subcore_barrier()
    pltpu.sync_copy(deg_sh.at[pl.ds(row0, _RPT)],
                    out_hbm.at[cid, pl.ds(row0, _RPT)])


def _sc_degree(dst_idx, zeros_w, ones_w):
    return pl.kernel(
        _sc_degree_body,
        out_type=jax.ShapeDtypeStruct((_NCORE, _NPAD, _DW), jnp.float32),
        mesh=_mesh,
        scratch_types=[
            pltpu.VMEM((_CHUNKS, _CHUNK), jnp.int32),
            pltpu.VMEM((_CHUNK, _DW), jnp.float32),
            pltpu.VMEM_SHARED((_NPAD, _DW), jnp.float32),
        ],
    )(dst_idx, zeros_w, ones_w)


def _sc_scatter_body(table_hbm, src_hbm, dst_hbm, zeros_hbm, out_hbm,
                     src_v, dst_v, rows_v, acc_sh, sem):
    cid = lax.axis_index("c")
    sid = lax.axis_index("s")
    wid = cid * _NSUB + sid
    row0 = sid * _RPT
    pltpu.sync_copy(zeros_hbm.at[pl.ds(row0, _RPT)],
                    acc_sh.at[pl.ds(row0, _RPT)])
    pltpu.sync_copy(src_hbm.at[wid], src_v)
    pltpu.sync_copy(dst_hbm.at[wid], dst_v)
    plsc.subcore_barrier()

    def step(j, carry):
        pltpu.async_copy(table_hbm.at[src_v.at[j]], rows_v, sem).wait()
        pltpu.sync_copy(rows_v, acc_sh.at[dst_v.at[j]], add=True)
        return carry

    lax.fori_loop(0, _CHUNKS, step, 0)
    plsc.subcore_barrier()
    pltpu.sync_copy(acc_sh.at[pl.ds(row0, _RPT)],
                    out_hbm.at[cid, pl.ds(row0, _RPT)])


def _sc_scatter(table, src_idx, dst_idx, zeros_f):
    return pl.kernel(
        _sc_scatter_body,
        out_type=jax.ShapeDtypeStruct((_NCORE, _NPAD, 128), jnp.float32),
        mesh=_mesh,
        scratch_types=[
            pltpu.VMEM((_CHUNKS, _CHUNK), jnp.int32),
            pltpu.VMEM((_CHUNKS, _CHUNK), jnp.int32),
            pltpu.VMEM((_CHUNK, 128), jnp.float32),
            pltpu.VMEM_SHARED((_NPAD, 128), jnp.float32),
            pltpu.SemaphoreType.DMA,
        ],
    )(table, src_idx, dst_idx, zeros_f)


_BR = 400
_GRID = _N // _BR


def _dinv_of(d0_ref, d1_ref):
    deg = d0_ref[:, :1] + d1_ref[:, :1] + 1.0
    return lax.rsqrt(deg)


def _tc_prep_body(d0_ref, d1_ref, x_ref, xs_ref):
    xs_ref[:, :] = x_ref[:, :] * _dinv_of(d0_ref, d1_ref)


def _tc_prep(d0, d1, x):
    return pl.pallas_call(
        _tc_prep_body,
        grid=(_GRID,),
        in_specs=[
            pl.BlockSpec((_BR, _DW), lambda i: (i, 0)),
            pl.BlockSpec((_BR, _DW), lambda i: (i, 0)),
            pl.BlockSpec((_BR, 128), lambda i: (i, 0)),
        ],
        out_specs=pl.BlockSpec((_BR, 128), lambda i: (i, 0)),
        out_shape=jax.ShapeDtypeStruct((_N, 128), jnp.float32),
    )(d0, d1, x)


def _tc_mm1_body(d0_ref, d1_ref, p0_ref, p1_ref, x_ref, w1_ref, b1_ref,
                 wc_ref, h2_ref, hs_ref):
    dinv = _dinv_of(d0_ref, d1_ref)
    agg = dinv * (p0_ref[:, :] + p1_ref[:, :]) + (dinv * dinv) * x_ref[:, :]
    hid = jnp.dot(agg, w1_ref[:, :], preferred_element_type=jnp.float32)
    hid = jnp.maximum(hid + b1_ref[:, :], 0.0)
    h2 = jnp.dot(hid, wc_ref[:, :], preferred_element_type=jnp.float32)
    h2_ref[:, :] = h2
    hs_ref[:, :] = h2 * dinv


def _tc_mm1(d0, d1, p0, p1, x, w1, b1, wc):
    return pl.pallas_call(
        _tc_mm1_body,
        grid=(_GRID,),
        in_specs=[
            pl.BlockSpec((_BR, _DW), lambda i: (i, 0)),
            pl.BlockSpec((_BR, _DW), lambda i: (i, 0)),
            pl.BlockSpec((_BR, 128), lambda i: (i, 0)),
            pl.BlockSpec((_BR, 128), lambda i: (i, 0)),
            pl.BlockSpec((_BR, 128), lambda i: (i, 0)),
            pl.BlockSpec((128, 256), lambda i: (0, 0)),
            pl.BlockSpec((1, 256), lambda i: (0, 0)),
            pl.BlockSpec((256, 128), lambda i: (0, 0)),
        ],
        out_specs=[
            pl.BlockSpec((_BR, 128), lambda i: (i, 0)),
            pl.BlockSpec((_BR, 128), lambda i: (i, 0)),
        ],
        out_shape=[
            jax.ShapeDtypeStruct((_N, 128), jnp.float32),
            jax.ShapeDtypeStruct((_N, 128), jnp.float32),
        ],
    )(d0, d1, p0, p1, x, w1, b1, wc)


def _tc_mm2_body(d0_ref, d1_ref, q0_ref, q1_ref, h2_ref, bc_ref, out_ref):
    dinv = _dinv_of(d0_ref, d1_ref)
    out_ref[:, :] = (dinv * (q0_ref[:, :] + q1_ref[:, :])
                     + (dinv * dinv) * h2_ref[:, :] + bc_ref[:, :])


def _tc_mm2(d0, d1, q0, q1, h2, bc):
    return pl.pallas_call(
        _tc_mm2_body,
        grid=(_GRID,),
        in_specs=[
            pl.BlockSpec((_BR, _DW), lambda i: (i, 0)),
            pl.BlockSpec((_BR, _DW), lambda i: (i, 0)),
            pl.BlockSpec((_BR, 128), lambda i: (i, 0)),
            pl.BlockSpec((_BR, 128), lambda i: (i, 0)),
            pl.BlockSpec((_BR, 128), lambda i: (i, 0)),
            pl.BlockSpec((1, 128), lambda i: (0, 0)),
        ],
        out_specs=pl.BlockSpec((_BR, 128), lambda i: (i, 0)),
        out_shape=jax.ShapeDtypeStruct((_N, 128), jnp.float32),
    )(d0, d1, q0, q1, h2, bc)


def _pad_rows(a):
    return jnp.concatenate(
        [a, jnp.zeros((_NPAD - _N, a.shape[1]), a.dtype)], axis=0)


@jax.jit
def kernel(x, edge_index, W1, b1, Wmu, bmu, Wls, bls):
    src = edge_index[0]
    dst = edge_index[1]
    pad = jnp.full((_EPAD - _E,), _N, dtype=jnp.int32)
    src3 = jnp.concatenate([src, pad]).reshape(_NW, _CHUNKS, _CHUNK)
    dst3 = jnp.concatenate([dst, pad]).reshape(_NW, _CHUNKS, _CHUNK)

    zeros_w = jnp.zeros((_NPAD, _DW), jnp.float32)
    ones_w = jnp.ones((_CHUNK, _DW), jnp.float32)
    zeros_f = jnp.zeros((_NPAD, 128), jnp.float32)

    degp = _sc_degree(dst3, zeros_w, ones_w)
    d0 = degp[0, :_N, :]
    d1 = degp[1, :_N, :]

    xs = _tc_prep(d0, d1, x)
    p = _sc_scatter(_pad_rows(xs), src3, dst3, zeros_f)

    wc = jnp.concatenate([Wmu, Wls], axis=1)
    bc = jnp.concatenate([bmu, bls]).reshape(1, 128)
    h2, hs = _tc_mm1(d0, d1, p[0, :_N, :], p[1, :_N, :], x,
                     W1, b1.reshape(1, 256), wc)

    q = _sc_scatter(_pad_rows(hs), src3, dst3, zeros_f)
    out2 = _tc_mm2(d0, d1, q[0, :_N, :], q[1, :_N, :], h2, bc)
    return out2[:, :64], out2[:, 64:]


# speedup vs baseline: 14.2515x; 1.0004x over previous
import functools

import jax
import jax.numpy as jnp
from jax import lax
from jax.experimental import pallas as pl
from jax.experimental.pallas import tpu as pltpu
from jax.experimental.pallas import tpu_sc as plsc

_N = 10000
_E = 320000
_NSUB = 16
_NCORE = 2
_NW = _NCORE * _NSUB
_CHUNK = 128
_CHUNKS = -(-_E // (_NW * _CHUNK))
_EPAD = _NW * _CHUNKS * _CHUNK
_NPAD = 10112
_RPT = _NPAD // _NSUB
_DW = 128

_mesh = plsc.VectorSubcoreMesh(core_axis_name="c", subcore_axis_name="s")


def _sc_degree_body(dst_hbm, zeros_hbm, ones_hbm, out_hbm,
                    dst_v, ones_v, deg_sh):
    cid = lax.axis_index("c")
    sid = lax.axis_index("s")
    wid = cid * _NSUB + sid
    row0 = sid * _RPT
    pltpu.sync_copy(zeros_hbm.at[pl.ds(row0, _RPT)],
                    deg_sh.at[pl.ds(row0, _RPT)])
    pltpu.sync_copy(dst_hbm.at[wid], dst_v)
    pltpu.sync_copy(ones_hbm, ones_v)
    plsc.subcore_barrier()

    def step(j, carry):
        pltpu.sync_copy(ones_v, deg_sh.at[dst_v.at[j]], add=True)
        return carry

    lax.fori_loop(0, _CHUNKS, step, 0)
    plsc.subcore_barrier()
    pltpu.sync_copy(deg_sh.at[pl.ds(row0, _RPT)],
                    out_hbm.at[cid, pl.ds(row0, _RPT)])


def _sc_degree(dst_idx, zeros_w, ones_w):
    return pl.kernel(
        _sc_degree_body,
        out_type=jax.ShapeDtypeStruct((_NCORE, _NPAD, _DW), jnp.float32),
        mesh=_mesh,
        scratch_types=[
            pltpu.VMEM((_CHUNKS, _CHUNK), jnp.int32),
            pltpu.VMEM((_CHUNK, _DW), jnp.float32),
            pltpu.VMEM_SHARED((_NPAD, _DW), jnp.float32),
        ],
    )(dst_idx, zeros_w, ones_w)


def _sc_scatter_body(table_hbm, src_hbm, dst_hbm, zeros_hbm, out_hbm,
                     src_v, dst_v, rows_a, acc_sh, sem_a):
    cid = lax.axis_index("c")
    sid = lax.axis_index("s")
    wid = cid * _NSUB + sid
    row0 = sid * _RPT
    pltpu.sync_copy(src_hbm.at[wid], src_v)
    pltpu.sync_copy(dst_hbm.at[wid], dst_v)
    pltpu.sync_copy(zeros_hbm.at[pl.ds(row0, _RPT)],
                    acc_sh.at[pl.ds(row0, _RPT)])
    plsc.subcore_barrier()

    def step(j, carry):
        pltpu.sync_copy(table_hbm.at[src_v.at[j]], rows_a)
        pltpu.sync_copy(rows_a, acc_sh.at[dst_v.at[j]], add=True)
        return carry

    lax.fori_loop(0, _CHUNKS, step, 0)
    plsc.subcore_barrier()
    pltpu.sync_copy(acc_sh.at[pl.ds(row0, _RPT)],
                    out_hbm.at[cid, pl.ds(row0, _RPT)])


def _sc_scatter(table, src_idx, dst_idx, zeros_f):
    return pl.kernel(
        _sc_scatter_body,
        out_type=jax.ShapeDtypeStruct((_NCORE, _NPAD, 128), jnp.float32),
        mesh=_mesh,
        scratch_types=[
            pltpu.VMEM((_CHUNKS, _CHUNK), jnp.int32),
            pltpu.VMEM((_CHUNKS, _CHUNK), jnp.int32),
            pltpu.VMEM((_CHUNK, 128), jnp.float32),
            pltpu.VMEM_SHARED((_NPAD, 128), jnp.float32),
            pltpu.SemaphoreType.DMA,
        ],
    )(table, src_idx, dst_idx, zeros_f)


_BR = 400
_GRID = _N // _BR


def _dinv_of(d0_ref, d1_ref):
    deg = d0_ref[:, :1] + d1_ref[:, :1] + 1.0
    return lax.rsqrt(deg)


def _tc_prep_body(d0_ref, d1_ref, x_ref, xs_ref):
    xs_ref[:, :] = x_ref[:, :] * _dinv_of(d0_ref, d1_ref)


def _tc_prep(d0, d1, x):
    return pl.pallas_call(
        _tc_prep_body,
        grid=(_GRID,),
        in_specs=[
            pl.BlockSpec((_BR, _DW), lambda i: (i, 0)),
            pl.BlockSpec((_BR, _DW), lambda i: (i, 0)),
            pl.BlockSpec((_BR, 128), lambda i: (i, 0)),
        ],
        out_specs=pl.BlockSpec((_BR, 128), lambda i: (i, 0)),
        out_shape=jax.ShapeDtypeStruct((_N, 128), jnp.float32),
    )(d0, d1, x)


def _tc_mm1_body(d0_ref, d1_ref, p0_ref, p1_ref, x_ref, w1_ref, b1_ref,
                 wc_ref, h2_ref, hs_ref):
    dinv = _dinv_of(d0_ref, d1_ref)
    agg = dinv * (p0_ref[:, :] + p1_ref[:, :]) + (dinv * dinv) * x_ref[:, :]
    hid = jnp.dot(agg, w1_ref[:, :], preferred_element_type=jnp.float32)
    hid = jnp.maximum(hid + b1_ref[:, :], 0.0)
    h2 = jnp.dot(hid, wc_ref[:, :], preferred_element_type=jnp.float32)
    h2_ref[:, :] = h2
    hs_ref[:, :] = h2 * dinv


def _tc_mm1(d0, d1, p0, p1, x, w1, b1, wc):
    return pl.pallas_call(
        _tc_mm1_body,
        grid=(_GRID,),
        in_specs=[
            pl.BlockSpec((_BR, _DW), lambda i: (i, 0)),
            pl.BlockSpec((_BR, _DW), lambda i: (i, 0)),
            pl.BlockSpec((_BR, 128), lambda i: (i, 0)),
            pl.BlockSpec((_BR, 128), lambda i: (i, 0)),
            pl.BlockSpec((_BR, 128), lambda i: (i, 0)),
            pl.BlockSpec((128, 256), lambda i: (0, 0)),
            pl.BlockSpec((1, 256), lambda i: (0, 0)),
            pl.BlockSpec((256, 128), lambda i: (0, 0)),
        ],
        out_specs=[
            pl.BlockSpec((_BR, 128), lambda i: (i, 0)),
            pl.BlockSpec((_BR, 128), lambda i: (i, 0)),
        ],
        out_shape=[
            jax.ShapeDtypeStruct((_N, 128), jnp.float32),
            jax.ShapeDtypeStruct((_N, 128), jnp.float32),
        ],
    )(d0, d1, p0, p1, x, w1, b1, wc)


def _tc_mm2_body(d0_ref, d1_ref, q0_ref, q1_ref, h2_ref, bc_ref, out_ref):
    dinv = _dinv_of(d0_ref, d1_ref)
    out_ref[:, :] = (dinv * (q0_ref[:, :] + q1_ref[:, :])
                     + (dinv * dinv) * h2_ref[:, :] + bc_ref[:, :])


def _tc_mm2(d0, d1, q0, q1, h2, bc):
    return pl.pallas_call(
        _tc_mm2_body,
        grid=(_GRID,),
        in_specs=[
            pl.BlockSpec((_BR, _DW), lambda i: (i, 0)),
            pl.BlockSpec((_BR, _DW), lambda i: (i, 0)),
            pl.BlockSpec((_BR, 128), lambda i: (i, 0)),
            pl.BlockSpec((_BR, 128), lambda i: (i, 0)),
            pl.BlockSpec((_BR, 128), lambda i: (i, 0)),
            pl.BlockSpec((1, 128), lambda i: (0, 0)),
        ],
        out_specs=pl.BlockSpec((_BR, 128), lambda i: (i, 0)),
        out_shape=jax.ShapeDtypeStruct((_N, 128), jnp.float32),
    )(d0, d1, q0, q1, h2, bc)


def _pad_rows(a):
    return jnp.concatenate(
        [a, jnp.zeros((_NPAD - _N, a.shape[1]), a.dtype)], axis=0)


@jax.jit
def kernel(x, edge_index, W1, b1, Wmu, bmu, Wls, bls):
    src = edge_index[0]
    dst = edge_index[1]
    pad = jnp.full((_EPAD - _E,), _N, dtype=jnp.int32)
    src3 = jnp.concatenate([src, pad]).reshape(_NW, _CHUNKS, _CHUNK)
    dst3 = jnp.concatenate([dst, pad]).reshape(_NW, _CHUNKS, _CHUNK)

    zeros_w = jnp.zeros((_NPAD, _DW), jnp.float32)
    ones_w = jnp.ones((_CHUNK, _DW), jnp.float32)
    zeros_f = jnp.zeros((_NPAD, 128), jnp.float32)

    degp = _sc_degree(dst3, zeros_w, ones_w)
    d0 = degp[0, :_N, :]
    d1 = degp[1, :_N, :]

    xs = _tc_prep(d0, d1, x)
    p = _sc_scatter(_pad_rows(xs), src3, dst3, zeros_f)

    wc = jnp.concatenate([Wmu, Wls], axis=1)
    bc = jnp.concatenate([bmu, bls]).reshape(1, 128)
    h2, hs = _tc_mm1(d0, d1, p[0, :_N, :], p[1, :_N, :], x,
                     W1, b1.reshape(1, 256), wc)

    q = _sc_scatter(_pad_rows(hs), src3, dst3, zeros_f)
    out2 = _tc_mm2(d0, d1, q[0, :_N, :], q[1, :_N, :], h2, bc)
    return out2[:, :64], out2[:, 64:]


# speedup vs baseline: 30.1819x; 2.1178x over previous
import functools

import jax
import jax.numpy as jnp
from jax import lax
from jax.experimental import pallas as pl
from jax.experimental.pallas import tpu as pltpu
from jax.experimental.pallas import tpu_sc as plsc

_N = 10000
_E = 320000
_NSUB = 16
_NCORE = 2
_NW = _NCORE * _NSUB
_CHUNK = 128
_BLK = 16
_CHUNKS = 80
_NBLK = _CHUNKS // _BLK
_EPAD = _NW * _CHUNKS * _CHUNK
_NPAD = 10112
_RPT = _NPAD // _NSUB
_DW = 128

_mesh = plsc.VectorSubcoreMesh(core_axis_name="c", subcore_axis_name="s")


def _sc_degree_body(dst_hbm, zeros_hbm, ones_hbm, out_hbm,
                    dst_v, ones_v, deg_sh):
    cid = lax.axis_index("c")
    sid = lax.axis_index("s")
    wid = cid * _NSUB + sid
    row0 = sid * _RPT
    pltpu.sync_copy(zeros_hbm.at[pl.ds(row0, _RPT)],
                    deg_sh.at[pl.ds(row0, _RPT)])
    pltpu.sync_copy(dst_hbm.at[wid], dst_v)
    pltpu.sync_copy(ones_hbm, ones_v)
    plsc.subcore_barrier()

    def step(j, carry):
        pltpu.sync_copy(ones_v, deg_sh.at[dst_v.at[j]], add=True)
        return carry

    lax.fori_loop(0, _CHUNKS, step, 0)
    plsc.subcore_barrier()
    pltpu.sync_copy(deg_sh.at[pl.ds(row0, _RPT)],
                    out_hbm.at[cid, pl.ds(row0, _RPT)])


def _sc_degree(dst_idx, zeros_w, ones_w):
    return pl.kernel(
        _sc_degree_body,
        out_type=jax.ShapeDtypeStruct((_NCORE, _NPAD, _DW), jnp.float32),
        mesh=_mesh,
        scratch_types=[
            pltpu.VMEM((_CHUNKS, _CHUNK), jnp.int32),
            pltpu.VMEM((_CHUNK, _DW), jnp.float32),
            pltpu.VMEM_SHARED((_NPAD, _DW), jnp.float32),
        ],
    )(dst_idx, zeros_w, ones_w)


def _sc_scatter_body(table_hbm, src_hbm, dst_hbm, zeros_hbm, out_hbm,
                     src_r, dst_r, rows_a, rows_b, acc_sh,
                     sem_a, sem_b, sem_i):
    cid = lax.axis_index("c")
    sid = lax.axis_index("s")
    wid = cid * _NSUB + sid
    row0 = sid * _RPT
    pltpu.sync_copy(src_hbm.at[wid, pl.ds(0, _BLK)], src_r.at[pl.ds(0, _BLK)])
    pltpu.sync_copy(dst_hbm.at[wid, pl.ds(0, _BLK)], dst_r.at[pl.ds(0, _BLK)])
    pltpu.sync_copy(zeros_hbm.at[pl.ds(row0, _RPT)],
                    acc_sh.at[pl.ds(row0, _RPT)])
    plsc.subcore_barrier()

    def block(b, carry):
        slot = (b % 2) * _BLK
        nslot = ((b + 1) % 2) * _BLK

        @pl.when(b > 0)
        def _():
            pltpu.make_async_copy(src_hbm.at[wid, pl.ds(b * _BLK, _BLK)],
                                  src_r.at[pl.ds(slot, _BLK)], sem_i).wait()
            pltpu.make_async_copy(dst_hbm.at[wid, pl.ds(b * _BLK, _BLK)],
                                  dst_r.at[pl.ds(slot, _BLK)], sem_i).wait()

        @pl.when(b + 1 < _NBLK)
        def _():
            pltpu.async_copy(src_hbm.at[wid, pl.ds((b + 1) * _BLK, _BLK)],
                             src_r.at[pl.ds(nslot, _BLK)], sem_i)
            pltpu.async_copy(dst_hbm.at[wid, pl.ds((b + 1) * _BLK, _BLK)],
                             dst_r.at[pl.ds(nslot, _BLK)], sem_i)

        pltpu.async_copy(table_hbm.at[src_r.at[slot]], rows_a, sem_a)

        def pair(p, c2):
            r = slot + 2 * p
            pltpu.async_copy(table_hbm.at[src_r.at[r + 1]], rows_b, sem_b)
            pltpu.make_async_copy(table_hbm.at[src_r.at[r]], rows_a,
                                  sem_a).wait()
            pltpu.sync_copy(rows_a, acc_sh.at[dst_r.at[r]], add=True)
            pltpu.async_copy(table_hbm.at[src_r.at[r + 2]], rows_a, sem_a)
            pltpu.make_async_copy(table_hbm.at[src_r.at[r + 1]], rows_b,
                                  sem_b).wait()
            pltpu.sync_copy(rows_b, acc_sh.at[dst_r.at[r + 1]], add=True)
            return c2

        lax.fori_loop(0, _BLK // 2 - 1, pair, 0)
        pltpu.async_copy(table_hbm.at[src_r.at[slot + _BLK - 1]], rows_b,
                         sem_b)
        pltpu.make_async_copy(table_hbm.at[src_r.at[slot + _BLK - 2]], rows_a,
                              sem_a).wait()
        pltpu.sync_copy(rows_a, acc_sh.at[dst_r.at[slot + _BLK - 2]], add=True)
        pltpu.make_async_copy(table_hbm.at[src_r.at[slot + _BLK - 1]], rows_b,
                              sem_b).wait()
        pltpu.sync_copy(rows_b, acc_sh.at[dst_r.at[slot + _BLK - 1]], add=True)
        return carry

    lax.fori_loop(0, _NBLK, block, 0)
    plsc.subcore_barrier()
    pltpu.sync_copy(acc_sh.at[pl.ds(row0, _RPT)],
                    out_hbm.at[cid, pl.ds(row0, _RPT)])


def _sc_scatter(table, src_idx, dst_idx, zeros_f):
    return pl.kernel(
        _sc_scatter_body,
        out_type=jax.ShapeDtypeStruct((_NCORE, _NPAD, 128), jnp.float32),
        mesh=_mesh,
        scratch_types=[
            pltpu.VMEM((2 * _BLK, _CHUNK), jnp.int32),
            pltpu.VMEM((2 * _BLK, _CHUNK), jnp.int32),
            pltpu.VMEM((_CHUNK, 128), jnp.float32),
            pltpu.VMEM((_CHUNK, 128), jnp.float32),
            pltpu.VMEM_SHARED((_NPAD, 128), jnp.float32),
            pltpu.SemaphoreType.DMA,
            pltpu.SemaphoreType.DMA,
            pltpu.SemaphoreType.DMA,
        ],
    )(table, src_idx, dst_idx, zeros_f)


_BR = 400
_GRID = _N // _BR


def _dinv_of(d0_ref, d1_ref):
    deg = d0_ref[:, :1] + d1_ref[:, :1] + 1.0
    return lax.rsqrt(deg)


def _tc_prep_body(d0_ref, d1_ref, x_ref, xs_ref):
    xs_ref[:, :] = x_ref[:, :] * _dinv_of(d0_ref, d1_ref)


def _tc_prep(d0, d1, x):
    return pl.pallas_call(
        _tc_prep_body,
        grid=(_GRID,),
        in_specs=[
            pl.BlockSpec((_BR, _DW), lambda i: (i, 0)),
            pl.BlockSpec((_BR, _DW), lambda i: (i, 0)),
            pl.BlockSpec((_BR, 128), lambda i: (i, 0)),
        ],
        out_specs=pl.BlockSpec((_BR, 128), lambda i: (i, 0)),
        out_shape=jax.ShapeDtypeStruct((_N, 128), jnp.float32),
    )(d0, d1, x)


def _tc_mm1_body(d0_ref, d1_ref, p0_ref, p1_ref, x_ref, w1_ref, b1_ref,
                 wc_ref, h2_ref, hs_ref):
    dinv = _dinv_of(d0_ref, d1_ref)
    agg = dinv * (p0_ref[:, :] + p1_ref[:, :]) + (dinv * dinv) * x_ref[:, :]
    hid = jnp.dot(agg, w1_ref[:, :], preferred_element_type=jnp.float32)
    hid = jnp.maximum(hid + b1_ref[:, :], 0.0)
    h2 = jnp.dot(hid, wc_ref[:, :], preferred_element_type=jnp.float32)
    h2_ref[:, :] = h2
    hs_ref[:, :] = h2 * dinv


def _tc_mm1(d0, d1, p0, p1, x, w1, b1, wc):
    return pl.pallas_call(
        _tc_mm1_body,
        grid=(_GRID,),
        in_specs=[
            pl.BlockSpec((_BR, _DW), lambda i: (i, 0)),
            pl.BlockSpec((_BR, _DW), lambda i: (i, 0)),
            pl.BlockSpec((_BR, 128), lambda i: (i, 0)),
            pl.BlockSpec((_BR, 128), lambda i: (i, 0)),
            pl.BlockSpec((_BR, 128), lambda i: (i, 0)),
            pl.BlockSpec((128, 256), lambda i: (0, 0)),
            pl.BlockSpec((1, 256), lambda i: (0, 0)),
            pl.BlockSpec((256, 128), lambda i: (0, 0)),
        ],
        out_specs=[
            pl.BlockSpec((_BR, 128), lambda i: (i, 0)),
            pl.BlockSpec((_BR, 128), lambda i: (i, 0)),
        ],
        out_shape=[
            jax.ShapeDtypeStruct((_N, 128), jnp.float32),
            jax.ShapeDtypeStruct((_N, 128), jnp.float32),
        ],
    )(d0, d1, p0, p1, x, w1, b1, wc)


def _tc_mm2_body(d0_ref, d1_ref, q0_ref, q1_ref, h2_ref, bc_ref, out_ref):
    dinv = _dinv_of(d0_ref, d1_ref)
    out_ref[:, :] = (dinv * (q0_ref[:, :] + q1_ref[:, :])
                     + (dinv * dinv) * h2_ref[:, :] + bc_ref[:, :])


def _tc_mm2(d0, d1, q0, q1, h2, bc):
    return pl.pallas_call(
        _tc_mm2_body,
        grid=(_GRID,),
        in_specs=[
            pl.BlockSpec((_BR, _DW), lambda i: (i, 0)),
            pl.BlockSpec((_BR, _DW), lambda i: (i, 0)),
            pl.BlockSpec((_BR, 128), lambda i: (i, 0)),
            pl.BlockSpec((_BR, 128), lambda i: (i, 0)),
            pl.BlockSpec((_BR, 128), lambda i: (i, 0)),
            pl.BlockSpec((1, 128), lambda i: (0, 0)),
        ],
        out_specs=pl.BlockSpec((_BR, 128), lambda i: (i, 0)),
        out_shape=jax.ShapeDtypeStruct((_N, 128), jnp.float32),
    )(d0, d1, q0, q1, h2, bc)


def _pad_rows(a):
    return jnp.concatenate(
        [a, jnp.zeros((_NPAD - _N, a.shape[1]), a.dtype)], axis=0)


@jax.jit
def kernel(x, edge_index, W1, b1, Wmu, bmu, Wls, bls):
    src = edge_index[0]
    dst = edge_index[1]
    pad = _N + (jnp.arange(_EPAD - _E, dtype=jnp.int32) % (_NPAD - _N))
    src3 = jnp.concatenate([src, pad]).reshape(_NW, _CHUNKS, _CHUNK)
    dst3 = jnp.concatenate([dst, pad]).reshape(_NW, _CHUNKS, _CHUNK)

    zeros_w = jnp.zeros((_NPAD, _DW), jnp.float32)
    ones_w = jnp.ones((_CHUNK, _DW), jnp.float32)
    zeros_f = jnp.zeros((_NPAD, 128), jnp.float32)

    degp = _sc_degree(dst3, zeros_w, ones_w)
    d0 = degp[0, :_N, :]
    d1 = degp[1, :_N, :]

    xs = _tc_prep(d0, d1, x)
    p = _sc_scatter(_pad_rows(xs), src3, dst3, zeros_f)

    wc = jnp.concatenate([Wmu, Wls], axis=1)
    bc = jnp.concatenate([bmu, bls]).reshape(1, 128)
    h2, hs = _tc_mm1(d0, d1, p[0, :_N, :], p[1, :_N, :], x,
                     W1, b1.reshape(1, 256), wc)

    q = _sc_scatter(_pad_rows(hs), src3, dst3, zeros_f)
    out2 = _tc_mm2(d0, d1, q[0, :_N, :], q[1, :_N, :], h2, bc)
    return out2[:, :64], out2[:, 64:]


# speedup vs baseline: 32.4274x; 1.0744x over previous
import functools

import jax
import jax.numpy as jnp
from jax import lax
from jax.experimental import pallas as pl
from jax.experimental.pallas import tpu as pltpu
from jax.experimental.pallas import tpu_sc as plsc

_N = 10000
_E = 320000
_NSUB = 16
_NCORE = 2
_NW = _NCORE * _NSUB
_CHUNK = 128
_BLK = 16
_CHUNKS = 80
_NBLK = _CHUNKS // _BLK
_EPAD = _NW * _CHUNKS * _CHUNK
_NPAD = 10112
_RPT = _NPAD // _NSUB
_DW = 128

_mesh = plsc.VectorSubcoreMesh(core_axis_name="c", subcore_axis_name="s")


def _sc_degree_body(dst_hbm, zeros_hbm, ones_hbm, out_hbm,
                    dst_v, ones_v, deg_sh):
    cid = lax.axis_index("c")
    sid = lax.axis_index("s")
    wid = cid * _NSUB + sid
    row0 = sid * _RPT
    pltpu.sync_copy(zeros_hbm.at[pl.ds(row0, _RPT)],
                    deg_sh.at[pl.ds(row0, _RPT)])
    pltpu.sync_copy(dst_hbm.at[wid], dst_v)
    pltpu.sync_copy(ones_hbm, ones_v)
    plsc.subcore_barrier()

    def step(j, carry):
        pltpu.sync_copy(ones_v, deg_sh.at[dst_v.at[j]], add=True)
        return carry

    lax.fori_loop(0, _CHUNKS, step, 0)
    plsc.subcore_barrier()
    pltpu.sync_copy(deg_sh.at[pl.ds(row0, _RPT)],
                    out_hbm.at[cid, pl.ds(row0, _RPT)])


def _sc_degree(dst_idx, zeros_w, ones_w):
    return pl.kernel(
        _sc_degree_body,
        out_type=jax.ShapeDtypeStruct((_NCORE, _NPAD, _DW), jnp.float32),
        mesh=_mesh,
        scratch_types=[
            pltpu.VMEM((_CHUNKS, _CHUNK), jnp.int32),
            pltpu.VMEM((_CHUNK, _DW), jnp.float32),
            pltpu.VMEM_SHARED((_NPAD, _DW), jnp.float32),
        ],
    )(dst_idx, zeros_w, ones_w)


def _sc_scatter_body(table_hbm, src_hbm, dst_hbm, zeros_hbm, out_hbm,
                     src_r, dst_r, rows_a, rows_b, acc_sh,
                     sem_a, sem_b, sem_i):
    cid = lax.axis_index("c")
    sid = lax.axis_index("s")
    wid = cid * _NSUB + sid
    row0 = sid * _RPT
    pltpu.sync_copy(src_hbm.at[wid, pl.ds(0, _BLK)], src_r.at[pl.ds(0, _BLK)])
    pltpu.sync_copy(dst_hbm.at[wid, pl.ds(0, _BLK)], dst_r.at[pl.ds(0, _BLK)])
    pltpu.sync_copy(zeros_hbm.at[pl.ds(row0, _RPT)],
                    acc_sh.at[pl.ds(row0, _RPT)])
    plsc.subcore_barrier()

    def block(b, carry):
        slot = (b % 2) * _BLK
        nslot = ((b + 1) % 2) * _BLK

        @pl.when(b > 0)
        def _():
            pltpu.make_async_copy(src_hbm.at[wid, pl.ds(b * _BLK, _BLK)],
                                  src_r.at[pl.ds(slot, _BLK)], sem_i).wait()
            pltpu.make_async_copy(dst_hbm.at[wid, pl.ds(b * _BLK, _BLK)],
                                  dst_r.at[pl.ds(slot, _BLK)], sem_i).wait()

        @pl.when(b + 1 < _NBLK)
        def _():
            pltpu.async_copy(src_hbm.at[wid, pl.ds((b + 1) * _BLK, _BLK)],
                             src_r.at[pl.ds(nslot, _BLK)], sem_i)
            pltpu.async_copy(dst_hbm.at[wid, pl.ds((b + 1) * _BLK, _BLK)],
                             dst_r.at[pl.ds(nslot, _BLK)], sem_i)

        pltpu.async_copy(table_hbm.at[src_r.at[slot]], rows_a, sem_a)

        def pair(p, c2):
            r = slot + 2 * p
            pltpu.async_copy(table_hbm.at[src_r.at[r + 1]], rows_b, sem_b)
            pltpu.make_async_copy(table_hbm.at[src_r.at[r]], rows_a,
                                  sem_a).wait()
            pltpu.sync_copy(rows_a, acc_sh.at[dst_r.at[r]], add=True)
            pltpu.async_copy(table_hbm.at[src_r.at[r + 2]], rows_a, sem_a)
            pltpu.make_async_copy(table_hbm.at[src_r.at[r + 1]], rows_b,
                                  sem_b).wait()
            pltpu.sync_copy(rows_b, acc_sh.at[dst_r.at[r + 1]], add=True)
            return c2

        lax.fori_loop(0, _BLK // 2 - 1, pair, 0)
        pltpu.async_copy(table_hbm.at[src_r.at[slot + _BLK - 1]], rows_b,
                         sem_b)
        pltpu.make_async_copy(table_hbm.at[src_r.at[slot + _BLK - 2]], rows_a,
                              sem_a).wait()
        pltpu.sync_copy(rows_a, acc_sh.at[dst_r.at[slot + _BLK - 2]], add=True)
        pltpu.make_async_copy(table_hbm.at[src_r.at[slot + _BLK - 1]], rows_b,
                              sem_b).wait()
        pltpu.sync_copy(rows_b, acc_sh.at[dst_r.at[slot + _BLK - 1]], add=True)
        return carry

    lax.fori_loop(0, _NBLK, block, 0)
    plsc.subcore_barrier()
    pltpu.sync_copy(acc_sh.at[pl.ds(row0, _RPT)],
                    out_hbm.at[cid, pl.ds(row0, _RPT)])


def _sc_scatter(table, src_idx, dst_idx, zeros_f):
    return pl.kernel(
        _sc_scatter_body,
        out_type=jax.ShapeDtypeStruct((_NCORE, _NPAD, 128), jnp.float32),
        mesh=_mesh,
        scratch_types=[
            pltpu.VMEM((2 * _BLK, _CHUNK), jnp.int32),
            pltpu.VMEM((2 * _BLK, _CHUNK), jnp.int32),
            pltpu.VMEM((_CHUNK, 128), jnp.float32),
            pltpu.VMEM((_CHUNK, 128), jnp.float32),
            pltpu.VMEM_SHARED((_NPAD, 128), jnp.float32),
            pltpu.SemaphoreType.DMA,
            pltpu.SemaphoreType.DMA,
            pltpu.SemaphoreType.DMA,
        ],
    )(table, src_idx, dst_idx, zeros_f)


_BR = 400
_GRID = _N // _BR


def _dinv_of(deg_ref):
    deg = deg_ref[0, :, :1] + deg_ref[1, :, :1] + 1.0
    return lax.rsqrt(deg)


def _tc_prep_body(deg_ref, x_ref, xs_ref):
    xs_ref[:, :] = x_ref[:, :] * _dinv_of(deg_ref)


def _tc_prep(degp, x):
    return pl.pallas_call(
        _tc_prep_body,
        grid=(_GRID,),
        in_specs=[
            pl.BlockSpec((2, _BR, _DW), lambda i: (0, i, 0)),
            pl.BlockSpec((_BR, 128), lambda i: (i, 0)),
        ],
        out_specs=pl.BlockSpec((_BR, 128), lambda i: (i, 0)),
        out_shape=jax.ShapeDtypeStruct((_NPAD, 128), jnp.float32),
    )(degp, x)


def _tc_mm1_body(deg_ref, p_ref, x_ref, w1_ref, b1_ref,
                 wc_ref, h2_ref, hs_ref):
    dinv = _dinv_of(deg_ref)
    agg = (dinv * (p_ref[0, :, :] + p_ref[1, :, :])
           + (dinv * dinv) * x_ref[:, :])
    hid = jnp.dot(agg, w1_ref[:, :], preferred_element_type=jnp.float32)
    hid = jnp.maximum(hid + b1_ref[:, :], 0.0)
    h2 = jnp.dot(hid, wc_ref[:, :], preferred_element_type=jnp.float32)
    h2_ref[:, :] = h2
    hs_ref[:, :] = h2 * dinv


def _tc_mm1(degp, p, x, w1, b1, wc):
    return pl.pallas_call(
        _tc_mm1_body,
        grid=(_GRID,),
        in_specs=[
            pl.BlockSpec((2, _BR, _DW), lambda i: (0, i, 0)),
            pl.BlockSpec((2, _BR, 128), lambda i: (0, i, 0)),
            pl.BlockSpec((_BR, 128), lambda i: (i, 0)),
            pl.BlockSpec((128, 256), lambda i: (0, 0)),
            pl.BlockSpec((1, 256), lambda i: (0, 0)),
            pl.BlockSpec((256, 128), lambda i: (0, 0)),
        ],
        out_specs=[
            pl.BlockSpec((_BR, 128), lambda i: (i, 0)),
            pl.BlockSpec((_BR, 128), lambda i: (i, 0)),
        ],
        out_shape=[
            jax.ShapeDtypeStruct((_N, 128), jnp.float32),
            jax.ShapeDtypeStruct((_NPAD, 128), jnp.float32),
        ],
    )(degp, p, x, w1, b1, wc)


def _tc_mm2_body(deg_ref, q_ref, h2_ref, bc_ref, out_ref):
    dinv = _dinv_of(deg_ref)
    out_ref[:, :] = (dinv * (q_ref[0, :, :] + q_ref[1, :, :])
                     + (dinv * dinv) * h2_ref[:, :] + bc_ref[:, :])


def _tc_mm2(degp, q, h2, bc):
    return pl.pallas_call(
        _tc_mm2_body,
        grid=(_GRID,),
        in_specs=[
            pl.BlockSpec((2, _BR, _DW), lambda i: (0, i, 0)),
            pl.BlockSpec((2, _BR, 128), lambda i: (0, i, 0)),
            pl.BlockSpec((_BR, 128), lambda i: (i, 0)),
            pl.BlockSpec((1, 128), lambda i: (0, 0)),
        ],
        out_specs=pl.BlockSpec((_BR, 128), lambda i: (i, 0)),
        out_shape=jax.ShapeDtypeStruct((_N, 128), jnp.float32),
    )(degp, q, h2, bc)


@jax.jit
def kernel(x, edge_index, W1, b1, Wmu, bmu, Wls, bls):
    src = edge_index[0]
    dst = edge_index[1]
    pad = _N + (jnp.arange(_EPAD - _E, dtype=jnp.int32) % (_NPAD - _N))
    src3 = jnp.concatenate([src, pad]).reshape(_NW, _CHUNKS, _CHUNK)
    dst3 = jnp.concatenate([dst, pad]).reshape(_NW, _CHUNKS, _CHUNK)

    zeros_w = jnp.zeros((_NPAD, _DW), jnp.float32)
    ones_w = jnp.ones((_CHUNK, _DW), jnp.float32)
    zeros_f = jnp.zeros((_NPAD, 128), jnp.float32)

    degp = _sc_degree(dst3, zeros_w, ones_w)
    xs = _tc_prep(degp, x)
    p = _sc_scatter(xs, src3, dst3, zeros_f)

    wc = jnp.concatenate([Wmu, Wls], axis=1)
    bc = jnp.concatenate([bmu, bls]).reshape(1, 128)
    h2, hs = _tc_mm1(degp, p, x, W1, b1.reshape(1, 256), wc)

    q = _sc_scatter(hs, src3, dst3, zeros_f)
    out2 = _tc_mm2(degp, q, h2, bc)
    return out2[:, :64], out2[:, 64:]
